# baseline (speedup 1.0000x reference)
import jax
import jax.numpy as jnp
from jax import lax
from jax.experimental import pallas as pl
from jax.experimental.pallas import tpu as pltpu
from jax.experimental.pallas import tpu_sc as plsc

N_NODES = 10000
N_EDGES = 320000
D = 128

NC = 2
NS = 16
NW = NC * NS
CHUNK = 96
NBUF = 3
RB = 6
ZCH = 64

N_PAD = ((N_NODES + NS * ZCH - 1) // (NS * ZCH)) * (NS * ZCH)
_CPW_MIN = (N_EDGES + NW * CHUNK - 1) // (NW * CHUNK)
CPW = -(-_CPW_MIN // RB) * RB
CPW0 = 3 * CPW // 2
CPW1 = 2 * CPW - CPW0
E_PAD = NS * (CPW0 + CPW1) * CHUNK
ROWS_PER_TILE = N_PAD // NS


GSPLIT = 1
GROWS = CHUNK // GSPLIT


def _sc_body(srcf, dstf, x_hbm, summed_hbm, deg_hbm, *rest):
    isrc_v = rest[:GSPLIT * RB]
    rest = rest[GSPLIT * RB:]
    idst_v = rest[:RB]
    rows_v = rest[RB:RB + NBUF]
    deg_v = rest[RB + NBUF]
    summed_sh = rest[RB + NBUF + 1]
    sems = rest[RB + NBUF + 2:]
    isems = sems[:RB]
    idsems = sems[RB:2 * RB]
    gsems = sems[2 * RB:2 * RB + NBUF]
    ssems = sems[2 * RB + NBUF:]

    c = lax.axis_index("c")
    s = lax.axis_index("s")
    wid = s * NC + c

    zeros16 = jnp.zeros((16,), jnp.float32)
    ones16 = jnp.ones((16,), jnp.float32)

    def _zdeg(i, _):
        deg_v[pl.ds(i * 16, 16)] = zeros16
        return 0
    lax.fori_loop(0, N_PAD // 16, _zdeg, 0)

    def _zrow(i, _):
        rows_v[0][i // (D // 16), pl.ds((i % (D // 16)) * 16, 16)] = zeros16
        return 0
    lax.fori_loop(0, ZCH * (D // 16), _zrow, 0)

    def _zsh(k, _):
        pltpu.sync_copy(rows_v[0].at[pl.ds(0, ZCH)],
                        summed_sh.at[pl.ds(s * ROWS_PER_TILE + k * ZCH, ZCH)])
        return 0
    lax.fori_loop(0, ROWS_PER_TILE // ZCH, _zsh, 0)

    plsc.subcore_barrier()

    def _make_ifire(ebase):
        def _ifire(i, slot):
            off = ebase + i * CHUNK
            for g in range(GSPLIT):
                pltpu.async_copy(srcf.at[pl.ds(off + g * GROWS, GROWS)],
                                 isrc_v[GSPLIT * slot + g], isems[slot])
            pltpu.async_copy(dstf.at[pl.ds(off, CHUNK)], idst_v[slot],
                             idsems[slot])
        return _ifire

    def _iwait_src(slot):
        for g in range(GSPLIT):
            pltpu.make_async_copy(srcf.at[pl.ds(0, GROWS)],
                                  isrc_v[GSPLIT * slot + g],
                                  isems[slot]).wait()

    def _iwait_dst(slot):
        pltpu.make_async_copy(dstf.at[pl.ds(0, CHUNK)], idst_v[slot],
                              idsems[slot]).wait()

    def _gfire(slot, b):
        for g in range(GSPLIT):
            pltpu.async_copy(x_hbm.at[isrc_v[GSPLIT * slot + g]],
                             rows_v[b].at[pl.ds(g * GROWS, GROWS)], gsems[b])

    def _gwait(b):
        for g in range(GSPLIT):
            pltpu.make_async_copy(x_hbm.at[isrc_v[g]],
                                  rows_v[b].at[pl.ds(g * GROWS, GROWS)],
                                  gsems[b]).wait()

    def _sfire(slot, b):
        pltpu.async_copy(rows_v[b], summed_sh.at[idst_v[slot]],
                         ssems[b], add=True)

    def _swait(b):
        pltpu.make_async_copy(rows_v[b], summed_sh.at[idst_v[0]],
                              ssems[b]).wait()

    def _count_deg(slot):
        for j in range(CHUNK // 16):
            idx16 = idst_v[slot][pl.ds(j * 16, 16)]
            plsc.addupdate_scatter(deg_v, [idx16], ones16)

    def _edge_pipeline(cpw, ebase):
        _ifire = _make_ifire(ebase)

        def _step(b, ifire_chunk=None, fire_gather=True):
            _gwait(b % NBUF)
            if fire_gather:
                _iwait_src((b + 2) % RB)
                _gfire((b + 2) % RB, (b + 2) % NBUF)
            _iwait_dst(b)
            _sfire(b, b % NBUF)
            _count_deg(b)
            _swait(b % NBUF)
            if ifire_chunk is not None:
                _ifire(ifire_chunk, b)

        for b in range(RB):
            _ifire(b, b)
        for b in range(2):
            _iwait_src(b)
            _gfire(b, b)

        for b in range(RB):
            _step(b, ifire_chunk=RB + b)

        def _edge(t, _):
            for b in range(RB):
                _step(b, ifire_chunk=t * RB + 2 * RB + b)
            return 0
        lax.fori_loop(0, cpw // RB - 2, _edge, 0)

        for b in range(RB):
            _step(b, fire_gather=(b < RB - 2))

    @pl.when(c == 0)
    def _():
        _edge_pipeline(CPW0, s * (CPW0 * CHUNK))

    @pl.when(c == 1)
    def _():
        _edge_pipeline(CPW1, NS * (CPW0 * CHUNK) + s * (CPW1 * CHUNK))

    pltpu.sync_copy(deg_v, deg_hbm.at[wid])

    plsc.subcore_barrier()

    def _dump(k, _):
        r0 = s * ROWS_PER_TILE + k * ZCH
        pltpu.sync_copy(summed_sh.at[pl.ds(r0, ZCH)],
                        summed_hbm.at[pl.ds(c * N_PAD + r0, ZCH)])
        return 0
    lax.fori_loop(0, ROWS_PER_TILE // ZCH, _dump, 0)


_sc_scatter = pl.kernel(
    _sc_body,
    out_type=(
        jax.ShapeDtypeStruct((NC * N_PAD, D), jnp.float32),
        jax.ShapeDtypeStruct((NW, N_PAD), jnp.float32),
    ),
    mesh=plsc.VectorSubcoreMesh(core_axis_name="c", subcore_axis_name="s"),
    scratch_types=(
        (pltpu.VMEM((GROWS,), jnp.int32),) * (GSPLIT * RB)
        + (pltpu.VMEM((CHUNK,), jnp.int32),) * RB
        + (pltpu.VMEM((CHUNK, D), jnp.float32),) * NBUF
        + (
            pltpu.VMEM((N_PAD,), jnp.float32),
            pltpu.VMEM_SHARED((N_PAD, D), jnp.float32),
        )
        + (pltpu.SemaphoreType.DMA,) * (2 * RB + 2 * NBUF)
    ),
    compiler_params=pltpu.CompilerParams(needs_layout_passes=False),
)


def _tc_body(x_ref, summed_ref, degt_ref, wst_ref, wnt_ref, bias_ref, out_ref):
    ssum = summed_ref[:N_PAD, :] + summed_ref[N_PAD:, :]
    deg = jnp.sum(degt_ref[...], axis=1, keepdims=True)
    neigh = ssum * (1.0 / jnp.maximum(deg, 1.0))
    out_ref[...] = (
        jnp.dot(x_ref[...], wst_ref[...], preferred_element_type=jnp.float32)
        + jnp.dot(neigh, wnt_ref[...], preferred_element_type=jnp.float32)
        + bias_ref[...]
    )


_tc_epilogue = pl.pallas_call(
    _tc_body,
    out_shape=jax.ShapeDtypeStruct((N_PAD, D), jnp.float32),
)


def kernel(x, edge_index, W_self, W_neigh, bias):
    src = edge_index[0].astype(jnp.int32)
    dst = edge_index[1].astype(jnp.int32)
    x_pad = jnp.pad(x, ((0, N_PAD - N_NODES), (0, 0)))
    src_p = jnp.pad(src, (0, E_PAD - N_EDGES), constant_values=N_NODES)
    dst_p = jnp.pad(dst, (0, E_PAD - N_EDGES), constant_values=N_NODES)

    summed, deg = _sc_scatter(src_p, dst_p, x_pad)
    out = _tc_epilogue(x_pad, summed, deg.T, W_self.T, W_neigh.T,
                       bias.reshape(1, D))
    return out[:N_NODES]


# speedup vs baseline: 1.2425x; 1.2425x over previous
import jax
import jax.numpy as jnp
from jax import lax
from jax.experimental import pallas as pl
from jax.experimental.pallas import tpu as pltpu
from jax.experimental.pallas import tpu_sc as plsc

N_NODES = 10000
N_EDGES = 320000
D = 128

NC = 2
NS = 16
NW = NC * NS
CHUNK = 128
NBUF = 2
RB = 4

N_PAD = ((N_NODES + NS * 8 - 1) // (NS * 8)) * (NS * 8)
_CPW_MIN = (N_EDGES + NW * CHUNK - 1) // (NW * CHUNK)
CPW = -(-_CPW_MIN // RB) * RB
CPW0 = 3 * CPW // 2
CPW1 = 2 * CPW - CPW0
N_CHUNKS = NS * (CPW0 + CPW1)
E_PAD = N_CHUNKS * CHUNK
ROWS_PER_TILE = N_PAD // NS


def _sc_body(edges, x_hbm, z_deg, z_rows, summed_hbm, deg_hbm, *rest):
    islot_v = rest[:RB]
    rows_v = rest[RB:RB + NBUF]
    deg_v = rest[RB + NBUF]
    summed_sh = rest[RB + NBUF + 1]
    sems = rest[RB + NBUF + 2:]
    isems = sems[:RB]
    gsems = sems[RB:RB + NBUF]
    ssems = sems[RB + NBUF:]

    c = lax.axis_index("c")
    s = lax.axis_index("s")
    wid = s * NC + c

    ones16 = jnp.ones((16,), jnp.float32)

    pltpu.sync_copy(z_deg, deg_v)
    pltpu.sync_copy(z_rows, summed_sh.at[pl.ds(s * ROWS_PER_TILE, ROWS_PER_TILE)])

    plsc.subcore_barrier()

    def _make_ifire(gbase):
        def _ifire(i, slot):
            off = (gbase + i) * (2 * CHUNK)
            pltpu.async_copy(edges.at[pl.ds(off, 2 * CHUNK)], islot_v[slot],
                             isems[slot])
        return _ifire

    def _iwait(slot):
        pltpu.make_async_copy(edges.at[pl.ds(0, 2 * CHUNK)], islot_v[slot],
                              isems[slot]).wait()

    def _gfire(slot, b):
        pltpu.async_copy(x_hbm.at[islot_v[slot].at[pl.ds(0, CHUNK)]],
                         rows_v[b], gsems[b])

    def _gwait(b):
        pltpu.make_async_copy(x_hbm.at[islot_v[0].at[pl.ds(0, CHUNK)]],
                              rows_v[b], gsems[b]).wait()

    def _sfire(slot, b):
        pltpu.async_copy(rows_v[b],
                         summed_sh.at[islot_v[slot].at[pl.ds(CHUNK, CHUNK)]],
                         ssems[b], add=True)

    def _swait(b):
        pltpu.make_async_copy(rows_v[b],
                              summed_sh.at[islot_v[0].at[pl.ds(CHUNK, CHUNK)]],
                              ssems[b]).wait()

    def _count_deg(slot):
        for j in range(CHUNK // 16):
            idx16 = islot_v[slot][pl.ds(CHUNK + j * 16, 16)]
            plsc.addupdate_scatter(deg_v, [idx16], ones16)

    def _edge_pipeline(cpw, gbase):
        _ifire = _make_ifire(gbase)

        def _step(b, ifire_chunk=None, fire_gather=True):
            _gwait(b % NBUF)
            _sfire(b, b % NBUF)
            _count_deg(b)
            _swait(b % NBUF)
            if ifire_chunk is not None:
                _ifire(ifire_chunk, b)
            if fire_gather:
                _iwait((b + 2) % RB)
                _gfire((b + 2) % RB, (b + 2) % NBUF)

        for b in range(RB):
            _ifire(b, b)
        for b in range(2):
            _iwait(b)
            _gfire(b, b)

        for b in range(RB):
            _step(b, ifire_chunk=RB + b)

        def _edge(t, _):
            for b in range(RB):
                _step(b, ifire_chunk=t * RB + 2 * RB + b)
            return 0
        lax.fori_loop(0, cpw // RB - 2, _edge, 0)

        for b in range(RB):
            _step(b, fire_gather=(b < RB - 2))

    @pl.when(c == 0)
    def _():
        _edge_pipeline(CPW0, s * CPW0)

    @pl.when(c == 1)
    def _():
        _edge_pipeline(CPW1, NS * CPW0 + s * CPW1)

    pltpu.sync_copy(deg_v, deg_hbm.at[wid])

    plsc.subcore_barrier()

    r0 = s * ROWS_PER_TILE
    pltpu.sync_copy(summed_sh.at[pl.ds(r0, ROWS_PER_TILE)],
                    summed_hbm.at[pl.ds(c * N_PAD + r0, ROWS_PER_TILE)])


_sc_scatter = pl.kernel(
    _sc_body,
    out_type=(
        jax.ShapeDtypeStruct((NC * N_PAD, D), jnp.float32),
        jax.ShapeDtypeStruct((NW, N_PAD), jnp.float32),
    ),
    mesh=plsc.VectorSubcoreMesh(core_axis_name="c", subcore_axis_name="s"),
    scratch_types=(
        (pltpu.VMEM((2 * CHUNK,), jnp.int32),) * RB
        + (pltpu.VMEM((CHUNK, D), jnp.float32),) * NBUF
        + (
            pltpu.VMEM((N_PAD,), jnp.float32),
            pltpu.VMEM_SHARED((N_PAD, D), jnp.float32),
        )
        + (pltpu.SemaphoreType.DMA,) * (RB + 2 * NBUF)
    ),
    compiler_params=pltpu.CompilerParams(needs_layout_passes=False),
)


def _tc_body(x_ref, summed_ref, degt_ref, wst_ref, wnt_ref, bias_ref, out_ref):
    ssum = summed_ref[:N_PAD, :] + summed_ref[N_PAD:, :]
    deg = jnp.sum(degt_ref[...], axis=1, keepdims=True)
    neigh = ssum * (1.0 / jnp.maximum(deg, 1.0))
    out_ref[...] = (
        jnp.dot(x_ref[...], wst_ref[...], preferred_element_type=jnp.float32)
        + jnp.dot(neigh, wnt_ref[...], preferred_element_type=jnp.float32)
        + bias_ref[...]
    )


_tc_epilogue = pl.pallas_call(
    _tc_body,
    out_shape=jax.ShapeDtypeStruct((N_PAD, D), jnp.float32),
)


def kernel(x, edge_index, W_self, W_neigh, bias):
    src = edge_index[0].astype(jnp.int32)
    dst = edge_index[1].astype(jnp.int32)
    x_pad = jnp.pad(x, ((0, N_PAD - N_NODES), (0, 0)))
    src_p = jnp.pad(src, (0, E_PAD - N_EDGES), constant_values=N_NODES)
    dst_p = jnp.pad(dst, (0, E_PAD - N_EDGES), constant_values=N_NODES)
    edges = jnp.stack(
        [src_p.reshape(N_CHUNKS, CHUNK), dst_p.reshape(N_CHUNKS, CHUNK)],
        axis=1).reshape(-1)
    z_deg = jnp.zeros((N_PAD,), jnp.float32)
    z_rows = jnp.zeros((ROWS_PER_TILE, D), jnp.float32)

    summed, deg = _sc_scatter(edges, x_pad, z_deg, z_rows)
    out = _tc_epilogue(x_pad, summed, deg.T, W_self.T, W_neigh.T,
                       bias.reshape(1, D))
    return out[:N_NODES]


# speedup vs baseline: 1.5934x; 1.2824x over previous
import jax
import jax.numpy as jnp
from jax import lax
from jax.experimental import pallas as pl
from jax.experimental.pallas import tpu as pltpu
from jax.experimental.pallas import tpu_sc as plsc

N_NODES = 10000
N_EDGES = 320000
D = 128

NC = 2
NS = 16
NW = NC * NS
CHUNK = 128
NBUF = 2
RB = 4

N_PAD = ((N_NODES + NS * 8 - 1) // (NS * 8)) * (NS * 8)
_CPW_MIN = (N_EDGES + NW * CHUNK - 1) // (NW * CHUNK)
CPW = -(-_CPW_MIN // RB) * RB
CPW0 = 3 * CPW // 2
CPW1 = 2 * CPW - CPW0
N_CHUNKS = NS * (CPW0 + CPW1)
E_PAD = N_CHUNKS * CHUNK
ROWS_PER_TILE = N_PAD // NS


def _sc_body(srcf, dstf, x_hbm, z_deg, z_rows, summed_hbm, deg_hbm, *rest):
    isrc_v = rest[:RB]
    idst_v = rest[RB:2 * RB]
    rows_v = rest[2 * RB:2 * RB + NBUF]
    deg_v = rest[2 * RB + NBUF]
    summed_sh = rest[2 * RB + NBUF + 1]
    sems = rest[2 * RB + NBUF + 2:]
    isems = sems[:RB]
    idsems = sems[RB:2 * RB]
    gsems = sems[2 * RB:2 * RB + NBUF]
    ssems = sems[2 * RB + NBUF:]

    c = lax.axis_index("c")
    s = lax.axis_index("s")
    wid = s * NC + c

    ones16 = jnp.ones((16,), jnp.float32)

    pltpu.sync_copy(z_deg, deg_v)
    pltpu.sync_copy(z_rows, summed_sh.at[pl.ds(s * ROWS_PER_TILE, ROWS_PER_TILE)])

    plsc.subcore_barrier()

    def _make_ifire(gbase):
        def _ifire(i, slot):
            off = (gbase + i) * CHUNK
            pltpu.async_copy(srcf.at[pl.ds(off, CHUNK)], isrc_v[slot],
                             isems[slot])
            pltpu.async_copy(dstf.at[pl.ds(off, CHUNK)], idst_v[slot],
                             idsems[slot])
        return _ifire

    def _iwait_src(slot):
        pltpu.make_async_copy(srcf.at[pl.ds(0, CHUNK)], isrc_v[slot],
                              isems[slot]).wait()

    def _iwait_dst(slot):
        pltpu.make_async_copy(dstf.at[pl.ds(0, CHUNK)], idst_v[slot],
                              idsems[slot]).wait()

    def _gfire(slot, b):
        pltpu.async_copy(x_hbm.at[isrc_v[slot]], rows_v[b], gsems[b])

    def _gwait(b):
        pltpu.make_async_copy(x_hbm.at[isrc_v[0]], rows_v[b], gsems[b]).wait()

    def _sfire(slot, b):
        pltpu.async_copy(rows_v[b], summed_sh.at[idst_v[slot]],
                         ssems[b], add=True)

    def _swait(b):
        pltpu.make_async_copy(rows_v[b], summed_sh.at[idst_v[0]],
                              ssems[b]).wait()

    def _count_deg(slot):
        for j in range(CHUNK // 16):
            idx16 = idst_v[slot][pl.ds(j * 16, 16)]
            plsc.addupdate_scatter(deg_v, [idx16], ones16)

    def _edge_pipeline(cpw, gbase):
        _ifire = _make_ifire(gbase)

        def _step(b, ifire_chunk=None, fire_gather=True):
            _iwait_dst(b)
            _gwait(b % NBUF)
            _sfire(b, b % NBUF)
            _count_deg(b)
            _swait(b % NBUF)
            if ifire_chunk is not None:
                _ifire(ifire_chunk, b)
            if fire_gather:
                _iwait_src((b + 2) % RB)
                _gfire((b + 2) % RB, (b + 2) % NBUF)

        for b in range(RB):
            _ifire(b, b)
        for b in range(2):
            _iwait_src(b)
            _gfire(b, b)

        for b in range(RB):
            _step(b, ifire_chunk=RB + b)

        def _edge(t, _):
            for b in range(RB):
                _step(b, ifire_chunk=t * RB + 2 * RB + b)
            return 0
        lax.fori_loop(0, cpw // RB - 2, _edge, 0)

        for b in range(RB):
            _step(b, fire_gather=(b < RB - 2))

    @pl.when(c == 0)
    def _():
        _edge_pipeline(CPW0, s * CPW0)

    @pl.when(c == 1)
    def _():
        _edge_pipeline(CPW1, NS * CPW0 + s * CPW1)

    pltpu.sync_copy(deg_v, deg_hbm.at[wid])

    plsc.subcore_barrier()

    r0 = s * ROWS_PER_TILE
    pltpu.sync_copy(summed_sh.at[pl.ds(r0, ROWS_PER_TILE)],
                    summed_hbm.at[pl.ds(c * N_PAD + r0, ROWS_PER_TILE)])


_sc_scatter = pl.kernel(
    _sc_body,
    out_type=(
        jax.ShapeDtypeStruct((NC * N_PAD, D), jnp.float32),
        jax.ShapeDtypeStruct((NW, N_PAD), jnp.float32),
    ),
    mesh=plsc.VectorSubcoreMesh(core_axis_name="c", subcore_axis_name="s"),
    scratch_types=(
        (pltpu.VMEM((CHUNK,), jnp.int32),) * (2 * RB)
        + (pltpu.VMEM((CHUNK, D), jnp.float32),) * NBUF
        + (
            pltpu.VMEM((N_PAD,), jnp.float32),
            pltpu.VMEM_SHARED((N_PAD, D), jnp.float32),
        )
        + (pltpu.SemaphoreType.DMA,) * (2 * RB + 2 * NBUF)
    ),
    compiler_params=pltpu.CompilerParams(needs_layout_passes=False),
)


def _tc_body(x_ref, summed_ref, degt_ref, wst_ref, wnt_ref, bias_ref, out_ref):
    ssum = summed_ref[:N_PAD, :] + summed_ref[N_PAD:, :]
    deg = jnp.sum(degt_ref[...], axis=1, keepdims=True)
    neigh = ssum * (1.0 / jnp.maximum(deg, 1.0))
    out_ref[...] = (
        jnp.dot(x_ref[...], wst_ref[...], preferred_element_type=jnp.float32)
        + jnp.dot(neigh, wnt_ref[...], preferred_element_type=jnp.float32)
        + bias_ref[...]
    )


_tc_epilogue = pl.pallas_call(
    _tc_body,
    out_shape=jax.ShapeDtypeStruct((N_PAD, D), jnp.float32),
)


def kernel(x, edge_index, W_self, W_neigh, bias):
    src = edge_index[0].astype(jnp.int32)
    dst = edge_index[1].astype(jnp.int32)
    x_pad = jnp.pad(x, ((0, N_PAD - N_NODES), (0, 0)))
    src_p = jnp.pad(src, (0, E_PAD - N_EDGES), constant_values=N_NODES)
    dst_p = jnp.pad(dst, (0, E_PAD - N_EDGES), constant_values=N_NODES)
    z_deg = jnp.zeros((N_PAD,), jnp.float32)
    z_rows = jnp.zeros((ROWS_PER_TILE, D), jnp.float32)

    summed, deg = _sc_scatter(src_p, dst_p, x_pad, z_deg, z_rows)
    out = _tc_epilogue(x_pad, summed, deg.T, W_self.T, W_neigh.T,
                       bias.reshape(1, D))
    return out[:N_NODES]
